# Initial kernel scaffold; baseline (speedup 1.0000x reference)
import jax
import jax.numpy as jnp
from jax.experimental import pallas as pl


def kernel(batch, labels):
    raise NotImplementedError("write your pallas kernel here")


# speedup vs baseline: 17.5254x; 17.5254x over previous
import jax
import jax.numpy as jnp
from jax.experimental import pallas as pl
from jax.experimental.pallas import tpu as pltpu

B = 4096
D = 128
BLK = 512
KPC = 4


def _bce_body(batch_ref, lbl_row_ref, lbl_col_ref, out_ref, bn_ref):
    k = pl.program_id(0)

    @pl.when(k == 0)
    def _init():
        b = batch_ref[...]
        nrm = jnp.sqrt(jnp.sum(b * b, axis=1, keepdims=True))
        bn_ref[...] = b / jnp.maximum(nrm, 1e-12)
        out_ref[...] = jnp.zeros((1, 1), jnp.float32)

    bn_rows = bn_ref[pl.ds(k * BLK, BLK), :]
    sim = jax.lax.dot_general(
        bn_rows, bn_ref[...], (((1,), (1,)), ((), ())),
        preferred_element_type=jnp.float32)
    e = jnp.exp(sim)

    same = lbl_row_ref[...] == lbl_col_ref[...]
    s_all = jnp.sum(e, axis=1, keepdims=True)
    t_same = jnp.sum(jnp.where(same, e, 0.0), axis=1, keepdims=True)
    loss2 = s_all - t_same

    gr = k * BLK + jax.lax.broadcasted_iota(jnp.int32, (BLK, B), 0)
    col = jax.lax.broadcasted_iota(jnp.int32, (BLK, B), 1)
    pos = (col == (gr // KPC) * KPC) & (gr % KPC != 0)
    loss1 = jnp.sum(jnp.where(pos, e, 0.0), axis=1, keepdims=True)

    per_row = jnp.log(loss2) - jnp.log(loss1)
    out_ref[...] += jnp.sum(per_row).reshape(1, 1)


def kernel(batch, labels):
    lbl_row = labels.reshape(B, 1)
    lbl_col = labels.reshape(1, B)
    total = pl.pallas_call(
        _bce_body,
        grid=(B // BLK,),
        in_specs=[
            pl.BlockSpec((B, D), lambda k: (0, 0)),
            pl.BlockSpec((BLK, 1), lambda k: (k, 0)),
            pl.BlockSpec((1, B), lambda k: (0, 0)),
        ],
        out_specs=pl.BlockSpec((1, 1), lambda k: (0, 0)),
        out_shape=jax.ShapeDtypeStruct((1, 1), jnp.float32),
        scratch_shapes=[pltpu.VMEM((B, D), jnp.float32)],
    )(batch, lbl_row, lbl_col)
    return total[0, 0] / B
